# Initial kernel scaffold; baseline (speedup 1.0000x reference)
import jax
import jax.numpy as jnp
from jax.experimental import pallas as pl


def kernel(x, table):
    raise NotImplementedError("write your pallas kernel here")


# speedup vs baseline: 1.3134x; 1.3134x over previous
import jax
import jax.numpy as jnp
from jax.experimental import pallas as pl

B, D, L = 4, 1024, 8192
DB = 512
LB = 1024


def _ape_add_body(x_ref, t_ref, o_ref):
    ape_t = jnp.transpose(t_ref[...], (1, 0))
    o_ref[...] = x_ref[...] + ape_t[None, :, :]


def kernel(x, table):
    table_l = table[:L]
    grid = (L // LB, D // DB, B)
    return pl.pallas_call(
        _ape_add_body,
        grid=grid,
        in_specs=[
            pl.BlockSpec((1, DB, LB), lambda l, d, b: (b, d, l)),
            pl.BlockSpec((LB, DB), lambda l, d, b: (l, d)),
        ],
        out_specs=pl.BlockSpec((1, DB, LB), lambda l, d, b: (b, d, l)),
        out_shape=jax.ShapeDtypeStruct((B, D, L), x.dtype),
    )(x, table_l)


# speedup vs baseline: 1.5301x; 1.1650x over previous
import jax
import jax.numpy as jnp
from jax.experimental import pallas as pl

B, D, L = 4, 1024, 8192
DB = 256


def _ape_add_body(x_ref, t_ref, o_ref):
    ape_t = jnp.transpose(t_ref[...], (1, 0))
    o_ref[...] = x_ref[...] + ape_t[None, :, :]


def kernel(x, table):
    table_l = table[:L]
    grid = (D // DB, B)
    return pl.pallas_call(
        _ape_add_body,
        grid=grid,
        in_specs=[
            pl.BlockSpec((1, DB, L), lambda d, b: (b, d, 0)),
            pl.BlockSpec((L, DB), lambda d, b: (0, d)),
        ],
        out_specs=pl.BlockSpec((1, DB, L), lambda d, b: (b, d, 0)),
        out_shape=jax.ShapeDtypeStruct((B, D, L), x.dtype),
    )(x, table_l)
